# baseline (speedup 1.0000x reference)
import functools

import jax
import jax.numpy as jnp
from jax import lax
from jax.experimental import pallas as pl
from jax.experimental.pallas import tpu as pltpu
from jax.experimental.pallas import tpu_sc as plsc

_N_GARMENT = 21
_N_INDEX = 10
_IDX_CHUNK = 128


def _sc_gather(table, idx):
    b = idx.shape[0]
    d = table.shape[1]
    info = plsc.get_sparse_core_info()
    nw = info.num_cores * info.num_subcores
    b_per_w = b // nw
    n_chunks = b_per_w // _IDX_CHUNK
    idx2 = idx.reshape(b // _IDX_CHUNK, _IDX_CHUNK)
    mesh = plsc.VectorSubcoreMesh(core_axis_name="c", subcore_axis_name="s")

    @functools.partial(
        pl.kernel,
        mesh=mesh,
        compiler_params=pltpu.CompilerParams(use_tc_tiling_on_sc=False),
        out_type=jax.ShapeDtypeStruct((b, d), jnp.float32),
        scratch_types=[
            pltpu.VMEM((n_chunks, _IDX_CHUNK), jnp.int32),
            pltpu.VMEM((b_per_w, d), jnp.float32),
            pltpu.SemaphoreType.DMA,
        ],
    )
    def gather_kernel(table_hbm, idx_hbm, out_hbm, idx_v, rows_v, sem):
        wid = lax.axis_index("s") * info.num_cores + lax.axis_index("c")
        pltpu.sync_copy(idx_hbm.at[pl.ds(wid * n_chunks, n_chunks)], idx_v)
        copies = [
            pltpu.async_copy(
                table_hbm.at[idx_v.at[j]],
                rows_v.at[pl.ds(j * _IDX_CHUNK, _IDX_CHUNK)],
                sem,
            )
            for j in range(n_chunks)
        ]
        for c in copies:
            c.wait()
        pltpu.sync_copy(rows_v, out_hbm.at[pl.ds(wid * b_per_w, b_per_w)])

    return gather_kernel(table, idx2)


def _mlp_body(emb_ref, g_ref, i_ref, w1_ref, b1_ref, w2_ref, b2_ref, o_ref):
    blk = emb_ref.shape[0]
    e = emb_ref[...]
    g = g_ref[...]
    i = i_ref[...]
    col = lax.broadcasted_iota(jnp.int32, (blk, 64), 1)
    oh = ((col == g) | (col == i + _N_GARMENT)).astype(jnp.float32)
    cc = jnp.concatenate([e, oh], axis=1)
    h = jnp.dot(cc, w1_ref[...], preferred_element_type=jnp.float32)
    h = jnp.maximum(h + b1_ref[...], 0.0)
    o = jnp.dot(h, w2_ref[...], preferred_element_type=jnp.float32)
    o_ref[...] = o + b2_ref[...]


def _mlp(emb, gid, iid, w1p, b1, w2, b2):
    b, d = emb.shape
    blk = 2048
    grid = b // blk
    return pl.pallas_call(
        _mlp_body,
        grid=(grid,),
        in_specs=[
            pl.BlockSpec((blk, d), lambda i: (i, 0)),
            pl.BlockSpec((blk, 1), lambda i: (i, 0)),
            pl.BlockSpec((blk, 1), lambda i: (i, 0)),
            pl.BlockSpec((128, d), lambda i: (0, 0)),
            pl.BlockSpec((1, d), lambda i: (0, 0)),
            pl.BlockSpec((d, d), lambda i: (0, 0)),
            pl.BlockSpec((1, d), lambda i: (0, 0)),
        ],
        out_specs=pl.BlockSpec((blk, d), lambda i: (i, 0)),
        out_shape=jax.ShapeDtypeStruct((b, d), jnp.float32),
    )(emb, gid.reshape(b, 1), iid.reshape(b, 1), w1p, b1.reshape(1, d),
      w2, b2.reshape(1, d))


def kernel(article_id, garment_group_name, index_group_name, emb_table,
           W1, b1, W2, b2):
    item_emb = _sc_gather(emb_table, article_id)
    w1p = jnp.zeros((128, W1.shape[1]), W1.dtype).at[: W1.shape[0]].set(W1)
    return _mlp(item_emb, garment_group_name, index_group_name, w1p, b1, W2, b2)
